# baseline (speedup 1.0000x reference)
import functools

import jax
import jax.numpy as jnp
from jax import lax
from jax.experimental import pallas as pl
from jax.experimental.pallas import tpu as pltpu
from jax.experimental.pallas import tpu_sc as plsc

B, S = 4, 2048
HIDDEN = 768
N_TOK = B * S
NC, NS = 2, 16
NW = NC * NS
STAGES = 2
STAGE_TOK = N_TOK // STAGES
TOK_PER_W = STAGE_TOK // NW
CHUNK = 64
N_CHUNKS = TOK_PER_W // CHUNK
NBUF = 2

_sc_mesh = plsc.VectorSubcoreMesh(
    core_axis_name="c", subcore_axis_name="s", num_cores=NC, num_subcores=NS
)


@functools.partial(
    pl.kernel,
    out_type=jax.ShapeDtypeStruct((STAGE_TOK, HIDDEN), jnp.float32),
    mesh=_sc_mesh,
    scratch_types=[
        pltpu.VMEM((N_CHUNKS, CHUNK), jnp.int32),
        pltpu.VMEM((NBUF, CHUNK, HIDDEN), jnp.float32),
        pltpu.SemaphoreType.DMA,
        pltpu.SemaphoreType.DMA,
    ],
)
def _sc_gather(ids_hbm, table_hbm, out_hbm, idx_v, rows_v, gsem, wsem):
    wid = lax.axis_index("s") * NC + lax.axis_index("c")
    base = wid * TOK_PER_W
    pltpu.sync_copy(ids_hbm.at[wid], idx_v)

    def gather(c, buf):
        return pltpu.make_async_copy(
            table_hbm.at[idx_v.at[c]], rows_v.at[buf], gsem
        )

    def writeback(c, buf):
        return pltpu.make_async_copy(
            rows_v.at[buf], out_hbm.at[pl.ds(base + c * CHUNK, CHUNK)], wsem
        )

    gather(0, 0).start()
    for c in range(N_CHUNKS):
        buf = c % NBUF
        gather(c, buf).wait()
        if c + 1 < N_CHUNKS:
            if c + 1 >= NBUF:
                writeback(c + 1 - NBUF, (c + 1) % NBUF).wait()
            gather(c + 1, (c + 1) % NBUF).start()
        writeback(c, buf).start()
    for c in range(max(0, N_CHUNKS - NBUF), N_CHUNKS):
        writeback(c, c % NBUF).wait()


TC_BLK = 2048
POS_BLKS = S // TC_BLK
STAGE_B = STAGE_TOK // S


def _tc_body(g_ref, pos_ref, tt_ref, seg_ref, gam_ref, bet_ref, out_ref):
    x = g_ref[...] + pos_ref[...]
    tt = tt_ref[...]
    seg = seg_ref[...]
    s0 = seg[0:1, :]
    x = x + s0 + tt * (seg[1:2, :] - s0)
    mean = jnp.mean(x, axis=-1, keepdims=True)
    xc = x - mean
    var = jnp.mean(xc * xc, axis=-1, keepdims=True)
    y = xc * lax.rsqrt(var + 1e-12)
    out_ref[...] = y * gam_ref[...] + bet_ref[...]


def _tc_ln(gathered, pos_table, ttf, seg_table, gamma2d, beta2d):
    return pl.pallas_call(
        _tc_body,
        grid=(POS_BLKS, STAGE_B),
        in_specs=[
            pl.BlockSpec((TC_BLK, HIDDEN), lambda j, b: (b * POS_BLKS + j, 0)),
            pl.BlockSpec((TC_BLK, HIDDEN), lambda j, b: (j, 0)),
            pl.BlockSpec((TC_BLK, 1), lambda j, b: (b * POS_BLKS + j, 0)),
            pl.BlockSpec((2, HIDDEN), lambda j, b: (0, 0)),
            pl.BlockSpec((1, HIDDEN), lambda j, b: (0, 0)),
            pl.BlockSpec((1, HIDDEN), lambda j, b: (0, 0)),
        ],
        out_specs=pl.BlockSpec((TC_BLK, HIDDEN), lambda j, b: (b * POS_BLKS + j, 0)),
        out_shape=jax.ShapeDtypeStruct((STAGE_TOK, HIDDEN), jnp.float32),
    )(gathered, pos_table, ttf, seg_table, gamma2d, beta2d)


def kernel(input_ids, token_type_ids, token_table, pos_table, seg_table, ln_gamma, ln_beta):
    ids_w = input_ids.reshape(STAGES, NW, N_CHUNKS, CHUNK).astype(jnp.int32)
    ttf = token_type_ids.reshape(STAGES, STAGE_TOK, 1).astype(jnp.float32)
    gamma2d = ln_gamma.reshape(1, HIDDEN)
    beta2d = ln_beta.reshape(1, HIDDEN)
    outs = []
    for s in range(STAGES):
        gathered = _sc_gather(ids_w[s], token_table)
        outs.append(
            _tc_ln(gathered, pos_table, ttf[s], seg_table, gamma2d, beta2d)
        )
    return jnp.concatenate(outs, axis=0).reshape(B, S, HIDDEN)


# speedup vs baseline: 1.1540x; 1.1540x over previous
import functools

import jax
import jax.numpy as jnp
from jax import lax
from jax.experimental import pallas as pl
from jax.experimental.pallas import tpu as pltpu
from jax.experimental.pallas import tpu_sc as plsc

B, S = 4, 2048
HIDDEN = 768
N_TOK = B * S
NC, NS = 2, 16
NW = NC * NS
STAGES = 4
POS_SLICE = S // STAGES
STAGE_TOK = B * POS_SLICE
TOK_PER_W = STAGE_TOK // NW
CHUNK = 32
N_CHUNKS = TOK_PER_W // CHUNK
NBUF = 2

_sc_mesh = plsc.VectorSubcoreMesh(
    core_axis_name="c", subcore_axis_name="s", num_cores=NC, num_subcores=NS
)


@functools.partial(
    pl.kernel,
    out_type=jax.ShapeDtypeStruct((STAGE_TOK, HIDDEN), jnp.float32),
    mesh=_sc_mesh,
    scratch_types=[
        pltpu.VMEM((N_CHUNKS, CHUNK), jnp.int32),
        pltpu.VMEM((NBUF, CHUNK, HIDDEN), jnp.float32),
        pltpu.SemaphoreType.DMA,
        pltpu.SemaphoreType.DMA,
    ],
)
def _sc_gather(ids_hbm, table_hbm, out_hbm, idx_v, rows_v, gsem, wsem):
    wid = lax.axis_index("s") * NC + lax.axis_index("c")
    base = wid * TOK_PER_W
    pltpu.sync_copy(ids_hbm.at[wid], idx_v)

    def gather(c):
        return pltpu.make_async_copy(
            table_hbm.at[idx_v.at[c]], rows_v.at[c % NBUF], gsem
        )

    def writeback(c):
        return pltpu.make_async_copy(
            rows_v.at[c % NBUF], out_hbm.at[pl.ds(base + c * CHUNK, CHUNK)], wsem
        )

    for c in range(min(NBUF, N_CHUNKS)):
        gather(c).start()
    for c in range(N_CHUNKS):
        gather(c).wait()
        writeback(c).start()
        if c + NBUF < N_CHUNKS:
            writeback(c).wait()
            gather(c + NBUF).start()
    for c in range(max(0, N_CHUNKS - NBUF), N_CHUNKS):
        writeback(c).wait()


def _tc_body(g_ref, pos_ref, tt_ref, seg_ref, gam_ref, bet_ref, out_ref):
    x = g_ref[...] + pos_ref[...]
    tt = tt_ref[...]
    seg = seg_ref[...]
    s0 = seg[0:1, :]
    x = x + s0 + tt * (seg[1:2, :] - s0)
    mean = jnp.mean(x, axis=-1, keepdims=True)
    xc = x - mean
    var = jnp.mean(xc * xc, axis=-1, keepdims=True)
    y = xc * lax.rsqrt(var + 1e-12)
    out_ref[...] = y * gam_ref[...] + bet_ref[...]


def _tc_body_acc(g_ref, pos_ref, tt_ref, seg_ref, gam_ref, bet_ref, acc_ref, out_ref):
    del acc_ref
    _tc_body(g_ref, pos_ref, tt_ref, seg_ref, gam_ref, bet_ref, out_ref)


def _tc_ln(s, gathered, pos_table, ttf, seg_table, gamma2d, beta2d, acc):
    in_specs = [
        pl.BlockSpec((POS_SLICE, HIDDEN), lambda b: (b, 0)),
        pl.BlockSpec((POS_SLICE, HIDDEN), lambda b: (s, 0)),
        pl.BlockSpec((POS_SLICE, 1), lambda b: (b, 0)),
        pl.BlockSpec((2, HIDDEN), lambda b: (0, 0)),
        pl.BlockSpec((1, HIDDEN), lambda b: (0, 0)),
        pl.BlockSpec((1, HIDDEN), lambda b: (0, 0)),
    ]
    args = [gathered, pos_table, ttf, seg_table, gamma2d, beta2d]
    body = _tc_body
    io_aliases = {}
    if acc is not None:
        in_specs.append(pl.BlockSpec(memory_space=pl.ANY))
        args.append(acc)
        body = _tc_body_acc
        io_aliases = {6: 0}
    return pl.pallas_call(
        body,
        grid=(B,),
        in_specs=in_specs,
        out_specs=pl.BlockSpec(
            (POS_SLICE, HIDDEN), lambda b: (b * STAGES + s, 0)
        ),
        out_shape=jax.ShapeDtypeStruct((N_TOK, HIDDEN), jnp.float32),
        input_output_aliases=io_aliases,
    )(*args)


def kernel(input_ids, token_type_ids, token_table, pos_table, seg_table, ln_gamma, ln_beta):
    ids32 = input_ids.astype(jnp.int32)
    ttf_all = token_type_ids.astype(jnp.float32)
    gamma2d = ln_gamma.reshape(1, HIDDEN)
    beta2d = ln_beta.reshape(1, HIDDEN)
    gathered = [
        _sc_gather(
            ids32[:, s * POS_SLICE:(s + 1) * POS_SLICE].reshape(NW, N_CHUNKS, CHUNK),
            token_table,
        )
        for s in range(STAGES)
    ]
    acc = None
    for s in range(STAGES):
        ttf = ttf_all[:, s * POS_SLICE:(s + 1) * POS_SLICE].reshape(STAGE_TOK, 1)
        acc = _tc_ln(s, gathered[s], pos_table, ttf, seg_table, gamma2d, beta2d, acc)
    return acc.reshape(B, S, HIDDEN)
